# baseline (speedup 1.0000x reference)
import functools

import jax
import jax.numpy as jnp
from jax import lax
from jax.experimental import pallas as pl
from jax.experimental.pallas import tpu as pltpu
from jax.experimental.pallas import tpu_sc as plsc

_TOP_K = 8
_LB_WEIGHT = 0.01


def _mlp_body(x_ref, W1_ref, b1_ref, W2_ref, b2_ref, W3_ref, b3_ref,
              scale_ref, lt_ref, loss_ref, usage_acc, *, n_total, grid_n):
    i = pl.program_id(0)
    E = W3_ref.shape[1]

    dot = functools.partial(
        jax.lax.dot_general,
        dimension_numbers=(((1,), (0,)), ((), ())),
        preferred_element_type=jnp.float32,
        precision=jax.lax.Precision.DEFAULT,
    )

    h = jnp.maximum(dot(x_ref[...], W1_ref[...]) + b1_ref[...], 0.0)
    h = jnp.maximum(dot(h, W2_ref[...]) + b2_ref[...], 0.0)
    logits = (dot(h, W3_ref[...]) + b3_ref[...]) * scale_ref[...]

    lt = logits.T
    lt_ref[...] = lt

    m = jnp.max(lt, axis=0, keepdims=True)
    e = jnp.exp(lt - m)
    p = e / jnp.sum(e, axis=0, keepdims=True)
    part = jnp.sum(p, axis=1).reshape(1, E)

    @pl.when(i == 0)
    def _init():
        usage_acc[...] = part

    @pl.when(i != 0)
    def _acc():
        usage_acc[...] += part

    @pl.when(i == grid_n - 1)
    def _loss():
        usage = usage_acc[...] / jnp.float32(n_total)
        u = jnp.float32(1.0 / E)
        kl = jnp.sum(u * (jnp.log(u) - jnp.log(usage + 1e-8))) / E
        loss_ref[0, 0] = kl * _LB_WEIGHT


def _mlp_logits(x, W1, b1, W2, b2, W3, b3, scale):
    N, D = x.shape
    H = W1.shape[1]
    E = W3.shape[1]
    BN = min(1024, N)
    grid_n = N // BN

    return pl.pallas_call(
        functools.partial(_mlp_body, n_total=N, grid_n=grid_n),
        grid=(grid_n,),
        in_specs=[
            pl.BlockSpec((BN, D), lambda i: (i, 0)),
            pl.BlockSpec((D, H), lambda i: (0, 0)),
            pl.BlockSpec((1, H), lambda i: (0, 0)),
            pl.BlockSpec((H, H), lambda i: (0, 0)),
            pl.BlockSpec((1, H), lambda i: (0, 0)),
            pl.BlockSpec((H, E), lambda i: (0, 0)),
            pl.BlockSpec((1, E), lambda i: (0, 0)),
            pl.BlockSpec((1, E), lambda i: (0, 0)),
        ],
        out_specs=[
            pl.BlockSpec((E, BN), lambda i: (0, i)),
            pl.BlockSpec(memory_space=pltpu.SMEM),
        ],
        out_shape=[
            jax.ShapeDtypeStruct((E, N), jnp.float32),
            jax.ShapeDtypeStruct((1, 1), jnp.float32),
        ],
        scratch_shapes=[pltpu.VMEM((1, E), jnp.float32)],
    )(x, W1, b1.reshape(1, H), W2, b2.reshape(1, H), W3, b3.reshape(1, E),
      scale)


def _sc_topk(logits_t):
    E, N = logits_t.shape
    info = plsc.get_sparse_core_info()
    NC, NS, L = info.num_cores, info.num_subcores, info.num_lanes
    NW = NC * NS
    TPW = N // NW
    G = TPW // L

    mesh = plsc.VectorSubcoreMesh(core_axis_name="c", subcore_axis_name="s")

    @functools.partial(
        pl.kernel,
        mesh=mesh,
        out_type=[
            jax.ShapeDtypeStruct((_TOP_K, N), jnp.float32),
            jax.ShapeDtypeStruct((_TOP_K, N), jnp.int32),
        ],
        scratch_types=[
            pltpu.VMEM((E, TPW), jnp.float32),
            pltpu.VMEM((_TOP_K, TPW), jnp.float32),
            pltpu.VMEM((_TOP_K, TPW), jnp.int32),
        ],
    )
    def route(lt_hbm, gates_hbm, idx_hbm, slab, gv, iv):
        wid = lax.axis_index("s") * NC + lax.axis_index("c")
        base = wid * TPW
        pltpu.sync_copy(lt_hbm.at[:, pl.ds(base, TPW)], slab)

        def one_group(off):
            neg = jnp.full((L,), -3.0e38, jnp.float32)
            zero = jnp.zeros((L,), jnp.int32)
            tv = [neg] * _TOP_K
            ti = [zero] * _TOP_K
            for e in range(E):
                v = slab[e, pl.ds(off, L)]
                vi = jnp.full((L,), e, jnp.int32)
                for k in range(_TOP_K):
                    gt = v > tv[k]
                    nv = jnp.where(gt, tv[k], v)
                    ni = jnp.where(gt, ti[k], vi)
                    tv[k] = jnp.where(gt, v, tv[k])
                    ti[k] = jnp.where(gt, vi, ti[k])
                    v, vi = nv, ni
            g = [jnp.exp(t - tv[0]) for t in tv]
            denom = g[0]
            for k in range(1, _TOP_K):
                denom = denom + g[k]
            for k in range(_TOP_K):
                gv[k, pl.ds(off, L)] = g[k] / denom
                iv[k, pl.ds(off, L)] = ti[k]

        def group(j, carry):
            one_group(j * (2 * L))
            one_group(j * (2 * L) + L)
            return carry

        lax.fori_loop(0, G // 2, group, 0)
        pltpu.sync_copy(gv, gates_hbm.at[:, pl.ds(base, TPW)])
        pltpu.sync_copy(iv, idx_hbm.at[:, pl.ds(base, TPW)])

    return route(logits_t)


def kernel(x, W1, b1, W2, b2, W3, b3, expert_importance, log_temperature):
    E = W3.shape[1]
    scale = (expert_importance * jnp.exp(-log_temperature)).reshape(1, E)
    logits_t, loss = _mlp_logits(x, W1, b1, W2, b2, W3, b3, scale)
    gates_t, idx_t = _sc_topk(logits_t)
    return gates_t.T, idx_t.T, loss.reshape(())


# speedup vs baseline: 1.1331x; 1.1331x over previous
import functools

import jax
import jax.numpy as jnp
from jax import lax
from jax.experimental import pallas as pl
from jax.experimental.pallas import tpu as pltpu
from jax.experimental.pallas import tpu_sc as plsc

_TOP_K = 8
_LB_WEIGHT = 0.01


def _mlp_body(x_ref, W1_ref, b1_ref, W2_ref, b2_ref, W3_ref, b3_ref,
              scale_ref, lt_ref, loss_ref, usage_acc, *, n_total, grid_n):
    i = pl.program_id(0)
    E = W3_ref.shape[1]

    dot = functools.partial(
        jax.lax.dot_general,
        dimension_numbers=(((1,), (0,)), ((), ())),
        preferred_element_type=jnp.float32,
        precision=jax.lax.Precision.DEFAULT,
    )

    h = jnp.maximum(dot(x_ref[...], W1_ref[...]) + b1_ref[...], 0.0)
    h = jnp.maximum(dot(h, W2_ref[...]) + b2_ref[...], 0.0)
    logits = (dot(h, W3_ref[...]) + b3_ref[...]) * scale_ref[...]

    lt = logits.T
    lt_ref[...] = lt

    m = jnp.max(lt, axis=0, keepdims=True)
    e = jnp.exp(lt - m)
    p = e / jnp.sum(e, axis=0, keepdims=True)
    part = jnp.sum(p, axis=1).reshape(1, E)

    @pl.when(i == 0)
    def _init():
        usage_acc[...] = part

    @pl.when(i != 0)
    def _acc():
        usage_acc[...] += part

    @pl.when(i == grid_n - 1)
    def _loss():
        usage = usage_acc[...] / jnp.float32(n_total)
        u = jnp.float32(1.0 / E)
        kl = jnp.sum(u * (jnp.log(u) - jnp.log(usage + 1e-8))) / E
        loss_ref[0, 0] = kl * _LB_WEIGHT


def _mlp_logits(x, W1, b1, W2, b2, W3, b3, scale):
    N, D = x.shape
    H = W1.shape[1]
    E = W3.shape[1]
    BN = min(1024, N)
    grid_n = N // BN

    return pl.pallas_call(
        functools.partial(_mlp_body, n_total=N, grid_n=grid_n),
        grid=(grid_n,),
        in_specs=[
            pl.BlockSpec((BN, D), lambda i: (i, 0)),
            pl.BlockSpec((D, H), lambda i: (0, 0)),
            pl.BlockSpec((1, H), lambda i: (0, 0)),
            pl.BlockSpec((H, H), lambda i: (0, 0)),
            pl.BlockSpec((1, H), lambda i: (0, 0)),
            pl.BlockSpec((H, E), lambda i: (0, 0)),
            pl.BlockSpec((1, E), lambda i: (0, 0)),
            pl.BlockSpec((1, E), lambda i: (0, 0)),
        ],
        out_specs=[
            pl.BlockSpec((E, BN), lambda i: (0, i)),
            pl.BlockSpec(memory_space=pltpu.SMEM),
        ],
        out_shape=[
            jax.ShapeDtypeStruct((E, N), jnp.float32),
            jax.ShapeDtypeStruct((1, 1), jnp.float32),
        ],
        scratch_shapes=[pltpu.VMEM((1, E), jnp.float32)],
    )(x, W1, b1.reshape(1, H), W2, b2.reshape(1, H), W3, b3.reshape(1, E),
      scale)


def _sc_topk(logits_t):
    E, N = logits_t.shape
    info = plsc.get_sparse_core_info()
    NC, NS, L = info.num_cores, info.num_subcores, info.num_lanes
    NW = NC * NS
    TPW = N // NW
    G = TPW // L

    mesh = plsc.VectorSubcoreMesh(core_axis_name="c", subcore_axis_name="s")

    @functools.partial(
        pl.kernel,
        mesh=mesh,
        out_type=[
            jax.ShapeDtypeStruct((_TOP_K, N), jnp.float32),
            jax.ShapeDtypeStruct((_TOP_K, N), jnp.int32),
        ],
        scratch_types=[
            pltpu.VMEM((E, TPW), jnp.float32),
            pltpu.VMEM((_TOP_K, TPW), jnp.float32),
            pltpu.VMEM((_TOP_K, TPW), jnp.int32),
        ],
    )
    def route(lt_hbm, gates_hbm, idx_hbm, slab, gv, iv):
        wid = lax.axis_index("s") * NC + lax.axis_index("c")
        base = wid * TPW
        pltpu.sync_copy(lt_hbm.at[:, pl.ds(base, TPW)], slab)

        def one_group(off):
            neg = jnp.full((L,), -3.0e38, jnp.float32)
            zero = jnp.zeros((L,), jnp.int32)
            tv = [neg] * _TOP_K
            ti = [zero] * _TOP_K
            for e in range(E):
                v = slab[e, pl.ds(off, L)]
                vi = jnp.full((L,), e, jnp.int32)
                for k in range(_TOP_K):
                    gt = v > tv[k]
                    hi = jnp.maximum(tv[k], v)
                    lo = jnp.minimum(tv[k], v)
                    ni = jnp.where(gt, ti[k], vi)
                    ti[k] = jnp.where(gt, vi, ti[k])
                    tv[k] = hi
                    v, vi = lo, ni
            g = [jnp.exp(t - tv[0]) for t in tv]
            denom = g[0]
            for k in range(1, _TOP_K):
                denom = denom + g[k]
            for k in range(_TOP_K):
                gv[k, pl.ds(off, L)] = g[k] / denom
                iv[k, pl.ds(off, L)] = ti[k]

        def group(j, carry):
            one_group(j * L)
            return carry

        lax.fori_loop(0, G, group, 0)
        pltpu.sync_copy(gv, gates_hbm.at[:, pl.ds(base, TPW)])
        pltpu.sync_copy(iv, idx_hbm.at[:, pl.ds(base, TPW)])

    return route(logits_t)


def kernel(x, W1, b1, W2, b2, W3, b3, expert_importance, log_temperature):
    E = W3.shape[1]
    scale = (expert_importance * jnp.exp(-log_temperature)).reshape(1, E)
    logits_t, loss = _mlp_logits(x, W1, b1, W2, b2, W3, b3, scale)
    gates_t, idx_t = _sc_topk(logits_t)
    return gates_t.T, idx_t.T, loss.reshape(())
